# baseline (speedup 1.0000x reference)
import functools
import numpy as np
import jax
import jax.numpy as jnp
from jax.experimental import pallas as pl
from jax.experimental.pallas import tpu as pltpu

_LEVELS = (8, 8, 8, 6, 5)
_CD = 5
_LANES = 128
_BM = 2048
_SUB = 128


def _fused(z_ref, win_ref, bin_ref, lo_ref, step_ref, inv_ref, maxi_ref,
           coefa_ref, coefb_ref, wout_ref, bout_ref,
           out_ref, idx_ref, loss_ref):
    lsum = jnp.zeros((1, 1), jnp.float32)
    for s in range(_BM // _SUB):
        rows = pl.ds(s * _SUB, _SUB)
        zp = jnp.dot(z_ref[rows, :], win_ref[...],
                     preferred_element_type=jnp.float32) + bin_ref[...]
        k = jnp.clip(jnp.round((zp - lo_ref[...]) * inv_ref[...]),
                     0.0, maxi_ref[...])
        q = lo_ref[...] + k * step_ref[...]
        e = zp - q
        lsum = lsum + jnp.sum(e * e).reshape(1, 1)
        idx_ref[rows, :] = jnp.sum(q * coefa_ref[...] + coefb_ref[...],
                                   axis=1, keepdims=True)
        acc = jnp.broadcast_to(bout_ref[...], (_SUB, out_ref.shape[1]))
        for i in range(_CD):
            acc = acc + q[:, i:i + 1] * wout_ref[i:i + 1, :]
        out_ref[rows, :] = acc
    loss_ref[...] = lsum.reshape(1, 1, 1)


def kernel(z, W_in, b_in, W_out, b_out, v0, v1, v2, v3, v4):
    b, n, dim = z.shape
    m = b * n
    cd = _CD
    nblk = m // _BM

    win_p = jnp.zeros((dim, _LANES), jnp.float32).at[:, :cd].set(W_in.T)
    wout_p = jnp.zeros((8, dim), jnp.float32).at[:cd, :].set(W_out.T)
    bin_p = jnp.zeros((1, _LANES), jnp.float32).at[0, :cd].set(b_in)
    bout_p = b_out.reshape(1, dim)

    vals = [np.linspace(-0.5, 0.5, lv).astype(np.float32) if lv % 2 else
            (np.arange(lv) / lv - 0.5).astype(np.float32)
            for lv in _LEVELS]
    lo_np = np.zeros((1, _LANES), np.float32)
    st_np = np.zeros((1, _LANES), np.float32)
    iv_np = np.zeros((1, _LANES), np.float32)
    mx_np = np.zeros((1, _LANES), np.float32)
    for i, v in enumerate(vals):
        lo_np[0, i] = v[0]
        st_np[0, i] = v[1] - v[0]
        iv_np[0, i] = 1.0 / (v[1] - v[0])
        mx_np[0, i] = _LEVELS[i] - 1
    basis = np.concatenate([[1], np.cumprod(_LEVELS[:-1])]).astype(np.int64)
    half = np.array(_LEVELS) // 2
    ca_np = np.zeros((1, _LANES), np.float32)
    cb_np = np.zeros((1, _LANES), np.float32)
    ca_np[0, :cd] = 2 * half * basis
    cb_np[0, :cd] = half * basis

    zf = z.reshape(m, dim)
    full = lambda i: (0, 0)
    out, idx, lpart = pl.pallas_call(
        _fused,
        grid=(nblk,),
        in_specs=[
            pl.BlockSpec((_BM, dim), lambda i: (i, 0)),
            pl.BlockSpec((dim, _LANES), full),
            pl.BlockSpec((1, _LANES), full),
            pl.BlockSpec((1, _LANES), full),
            pl.BlockSpec((1, _LANES), full),
            pl.BlockSpec((1, _LANES), full),
            pl.BlockSpec((1, _LANES), full),
            pl.BlockSpec((1, _LANES), full),
            pl.BlockSpec((1, _LANES), full),
            pl.BlockSpec((8, dim), full),
            pl.BlockSpec((1, dim), full),
        ],
        out_specs=[
            pl.BlockSpec((_BM, dim), lambda i: (i, 0)),
            pl.BlockSpec((_BM, 1), lambda i: (i, 0)),
            pl.BlockSpec((1, 1, 1), lambda i: (i, 0, 0)),
        ],
        out_shape=[
            jax.ShapeDtypeStruct((m, dim), jnp.float32),
            jax.ShapeDtypeStruct((m, 1), jnp.float32),
            jax.ShapeDtypeStruct((nblk, 1, 1), jnp.float32),
        ],
        compiler_params=pltpu.CompilerParams(
            dimension_semantics=("parallel",)),
    )(zf, win_p, bin_p, jnp.asarray(lo_np), jnp.asarray(st_np),
      jnp.asarray(iv_np), jnp.asarray(mx_np), jnp.asarray(ca_np),
      jnp.asarray(cb_np), wout_p, bout_p)

    out = out.reshape(b, n, dim)
    indices = idx.reshape(b, n)
    loss_val = jnp.sum(lpart) * (0.2 / (m * cd))
    return out, indices, loss_val


# speedup vs baseline: 1.0433x; 1.0433x over previous
import functools
import numpy as np
import jax
import jax.numpy as jnp
from jax.experimental import pallas as pl
from jax.experimental.pallas import tpu as pltpu

_LEVELS = (8, 8, 8, 6, 5)
_CD = 5
_LANES = 128
_BM = 2048
_SUB = 256


def _fused(z_ref, win_ref, bin_ref, lo_ref, step_ref, inv_ref, maxi_ref,
           coefa_ref, coefb_ref, wout_ref, bout_ref,
           out_ref, idx_ref, loss_ref):
    lsum = jnp.zeros((1, 1), jnp.float32)
    for s in range(_BM // _SUB):
        rows = pl.ds(s * _SUB, _SUB)
        zp = jnp.dot(z_ref[rows, :], win_ref[...],
                     preferred_element_type=jnp.float32) + bin_ref[...]
        k = jnp.clip(jnp.round((zp - lo_ref[...]) * inv_ref[...]),
                     0.0, maxi_ref[...])
        q = lo_ref[...] + k * step_ref[...]
        e = zp - q
        lsum = lsum + jnp.sum(e * e).reshape(1, 1)
        idx_ref[rows, :] = jnp.sum(q * coefa_ref[...] + coefb_ref[...],
                                   axis=1, keepdims=True)
        qb = [jnp.broadcast_to(q[:, i:i + 1], (_SUB, _LANES))
              for i in range(_CD)]
        for c in range(out_ref.shape[1] // _LANES):
            cols = pl.ds(c * _LANES, _LANES)
            acc = jnp.broadcast_to(bout_ref[0:1, cols], (_SUB, _LANES))
            for i in range(_CD):
                acc = acc + qb[i] * wout_ref[i:i + 1, cols]
            out_ref[rows, cols] = acc
    loss_ref[...] = lsum.reshape(1, 1, 1)


def kernel(z, W_in, b_in, W_out, b_out, v0, v1, v2, v3, v4):
    b, n, dim = z.shape
    m = b * n
    cd = _CD
    nblk = m // _BM

    win_p = jnp.zeros((dim, _LANES), jnp.float32).at[:, :cd].set(W_in.T)
    wout_p = jnp.zeros((8, dim), jnp.float32).at[:cd, :].set(W_out.T)
    bin_p = jnp.zeros((1, _LANES), jnp.float32).at[0, :cd].set(b_in)
    bout_p = b_out.reshape(1, dim)

    vals = [np.linspace(-0.5, 0.5, lv).astype(np.float32) if lv % 2 else
            (np.arange(lv) / lv - 0.5).astype(np.float32)
            for lv in _LEVELS]
    lo_np = np.zeros((1, _LANES), np.float32)
    st_np = np.zeros((1, _LANES), np.float32)
    iv_np = np.zeros((1, _LANES), np.float32)
    mx_np = np.zeros((1, _LANES), np.float32)
    for i, v in enumerate(vals):
        lo_np[0, i] = v[0]
        st_np[0, i] = v[1] - v[0]
        iv_np[0, i] = 1.0 / (v[1] - v[0])
        mx_np[0, i] = _LEVELS[i] - 1
    basis = np.concatenate([[1], np.cumprod(_LEVELS[:-1])]).astype(np.int64)
    half = np.array(_LEVELS) // 2
    ca_np = np.zeros((1, _LANES), np.float32)
    cb_np = np.zeros((1, _LANES), np.float32)
    ca_np[0, :cd] = 2 * half * basis
    cb_np[0, :cd] = half * basis

    zf = z.reshape(m, dim)
    full = lambda i: (0, 0)
    out, idx, lpart = pl.pallas_call(
        _fused,
        grid=(nblk,),
        in_specs=[
            pl.BlockSpec((_BM, dim), lambda i: (i, 0)),
            pl.BlockSpec((dim, _LANES), full),
            pl.BlockSpec((1, _LANES), full),
            pl.BlockSpec((1, _LANES), full),
            pl.BlockSpec((1, _LANES), full),
            pl.BlockSpec((1, _LANES), full),
            pl.BlockSpec((1, _LANES), full),
            pl.BlockSpec((1, _LANES), full),
            pl.BlockSpec((1, _LANES), full),
            pl.BlockSpec((8, dim), full),
            pl.BlockSpec((1, dim), full),
        ],
        out_specs=[
            pl.BlockSpec((_BM, dim), lambda i: (i, 0)),
            pl.BlockSpec((_BM, 1), lambda i: (i, 0)),
            pl.BlockSpec((1, 1, 1), lambda i: (i, 0, 0)),
        ],
        out_shape=[
            jax.ShapeDtypeStruct((m, dim), jnp.float32),
            jax.ShapeDtypeStruct((m, 1), jnp.float32),
            jax.ShapeDtypeStruct((nblk, 1, 1), jnp.float32),
        ],
        compiler_params=pltpu.CompilerParams(
            dimension_semantics=("parallel",)),
    )(zf, win_p, bin_p, jnp.asarray(lo_np), jnp.asarray(st_np),
      jnp.asarray(iv_np), jnp.asarray(mx_np), jnp.asarray(ca_np),
      jnp.asarray(cb_np), wout_p, bout_p)

    out = out.reshape(b, n, dim)
    indices = idx.reshape(b, n)
    loss_val = jnp.sum(lpart) * (0.2 / (m * cd))
    return out, indices, loss_val


# speedup vs baseline: 1.0461x; 1.0027x over previous
import functools
import numpy as np
import jax
import jax.numpy as jnp
from jax.experimental import pallas as pl
from jax.experimental.pallas import tpu as pltpu

_LEVELS = (8, 8, 8, 6, 5)
_CD = 5
_LANES = 128
_BM = 2048
_SUB = 256


def _fused(z_ref, win_ref, bin_ref, lo_ref, step_ref, inv_ref, maxi_ref,
           coefa_ref, coefb_ref, wout_ref, bout_ref,
           out_ref, idx_ref, loss_ref):
    lsum = jnp.zeros((1, 1), jnp.float32)
    for s in range(_BM // _SUB):
        rows = pl.ds(s * _SUB, _SUB)
        zp = jnp.dot(z_ref[rows, :], win_ref[...],
                     preferred_element_type=jnp.float32) + bin_ref[...]
        k = jnp.clip(jnp.round((zp - lo_ref[...]) * inv_ref[...]),
                     0.0, maxi_ref[...])
        q = lo_ref[...] + k * step_ref[...]
        e = zp - q
        lsum = lsum + jnp.sum(e * e).reshape(1, 1)
        idxcol = jnp.sum(q * coefa_ref[...] + coefb_ref[...],
                         axis=1, keepdims=True)
        idx_ref[s:s + 1, :] = idxcol.reshape(1, _SUB)
        qb = [jnp.broadcast_to(q[:, i:i + 1], (_SUB, _LANES))
              for i in range(_CD)]
        for c in range(out_ref.shape[1] // _LANES):
            cols = pl.ds(c * _LANES, _LANES)
            acc = jnp.broadcast_to(bout_ref[0:1, cols], (_SUB, _LANES))
            for i in range(_CD):
                acc = acc + qb[i] * wout_ref[i:i + 1, cols]
            out_ref[rows, cols] = acc
    loss_ref[...] = lsum.reshape(1, 1, 1)


def kernel(z, W_in, b_in, W_out, b_out, v0, v1, v2, v3, v4):
    b, n, dim = z.shape
    m = b * n
    cd = _CD
    nblk = m // _BM

    win_p = jnp.zeros((dim, _LANES), jnp.float32).at[:, :cd].set(W_in.T)
    wout_p = jnp.zeros((8, dim), jnp.float32).at[:cd, :].set(W_out.T)
    bin_p = jnp.zeros((1, _LANES), jnp.float32).at[0, :cd].set(b_in)
    bout_p = b_out.reshape(1, dim)

    vals = [np.linspace(-0.5, 0.5, lv).astype(np.float32) if lv % 2 else
            (np.arange(lv) / lv - 0.5).astype(np.float32)
            for lv in _LEVELS]
    lo_np = np.zeros((1, _LANES), np.float32)
    st_np = np.zeros((1, _LANES), np.float32)
    iv_np = np.zeros((1, _LANES), np.float32)
    mx_np = np.zeros((1, _LANES), np.float32)
    for i, v in enumerate(vals):
        lo_np[0, i] = v[0]
        st_np[0, i] = v[1] - v[0]
        iv_np[0, i] = 1.0 / (v[1] - v[0])
        mx_np[0, i] = _LEVELS[i] - 1
    basis = np.concatenate([[1], np.cumprod(_LEVELS[:-1])]).astype(np.int64)
    half = np.array(_LEVELS) // 2
    ca_np = np.zeros((1, _LANES), np.float32)
    cb_np = np.zeros((1, _LANES), np.float32)
    ca_np[0, :cd] = 2 * half * basis
    cb_np[0, :cd] = half * basis

    zf = z.reshape(m, dim)
    full = lambda i: (0, 0)
    out, idx, lpart = pl.pallas_call(
        _fused,
        grid=(nblk,),
        in_specs=[
            pl.BlockSpec((_BM, dim), lambda i: (i, 0)),
            pl.BlockSpec((dim, _LANES), full),
            pl.BlockSpec((1, _LANES), full),
            pl.BlockSpec((1, _LANES), full),
            pl.BlockSpec((1, _LANES), full),
            pl.BlockSpec((1, _LANES), full),
            pl.BlockSpec((1, _LANES), full),
            pl.BlockSpec((1, _LANES), full),
            pl.BlockSpec((1, _LANES), full),
            pl.BlockSpec((8, dim), full),
            pl.BlockSpec((1, dim), full),
        ],
        out_specs=[
            pl.BlockSpec((_BM, dim), lambda i: (i, 0)),
            pl.BlockSpec((_BM // _SUB, _SUB), lambda i: (i, 0)),
            pl.BlockSpec((1, 1, 1), lambda i: (i, 0, 0)),
        ],
        out_shape=[
            jax.ShapeDtypeStruct((m, dim), jnp.float32),
            jax.ShapeDtypeStruct((m // _SUB, _SUB), jnp.float32),
            jax.ShapeDtypeStruct((nblk, 1, 1), jnp.float32),
        ],
        compiler_params=pltpu.CompilerParams(
            dimension_semantics=("parallel",)),
    )(zf, win_p, bin_p, jnp.asarray(lo_np), jnp.asarray(st_np),
      jnp.asarray(iv_np), jnp.asarray(mx_np), jnp.asarray(ca_np),
      jnp.asarray(cb_np), wout_p, bout_p)

    out = out.reshape(b, n, dim)
    indices = idx.reshape(b, n)
    loss_val = jnp.sum(lpart) * (0.2 / (m * cd))
    return out, indices, loss_val
